# baseline (speedup 1.0000x reference)
import functools

import jax
import jax.numpy as jnp
from jax import lax
from jax.experimental import pallas as pl
from jax.experimental.pallas import tpu as pltpu
from jax.experimental.pallas import tpu_sc as plsc

NC = 2
NS = 16
NW = NC * NS
LANES = 16

C = 128
NBUF = 3


def _build_sc_kernel(e_pad: int, n_nodes: int, dim: int):
    assert dim == 64
    nch = e_pad // (NW * C)
    assert nch % NBUF == 0
    mesh = plsc.VectorSubcoreMesh(core_axis_name="c", subcore_axis_name="s")

    @functools.partial(
        pl.kernel,
        out_type=jax.ShapeDtypeStruct((NW * nch, C), jnp.float32),
        mesh=mesh,
        compiler_params=pltpu.CompilerParams(
            needs_layout_passes=False, use_tc_tiling_on_sc=False),
        scratch_types=[
            pltpu.VMEM((nch, C), jnp.int32),
            pltpu.VMEM((nch, C), jnp.int32),
        ] + [pltpu.VMEM((C, 64), jnp.float32) for _ in range(2 * NBUF)]
          + [pltpu.VMEM((16, LANES), jnp.float32),
             pltpu.VMEM((nch, C), jnp.float32)]
          + [pltpu.SemaphoreType.DMA for _ in range(2 * NBUF)],
    )
    def k(src_hbm, dst_hbm, emb_hbm, out_hbm, sidx, didx, *rest):
        rowbufs = rest[:2 * NBUF]
        part, outbuf = rest[2 * NBUF], rest[2 * NBUF + 1]
        sembufs = rest[2 * NBUF + 2:]
        wid = lax.axis_index("s") * NC + lax.axis_index("c")
        lane = lax.iota(jnp.int32, LANES)
        srows = tuple(rowbufs[2 * b] for b in range(NBUF))
        trows = tuple(rowbufs[2 * b + 1] for b in range(NBUF))
        sems = tuple((sembufs[2 * b], sembufs[2 * b + 1]) for b in range(NBUF))

        pltpu.sync_copy(src_hbm.at[pl.ds(wid * nch, nch), :], sidx)
        pltpu.sync_copy(dst_hbm.at[pl.ds(wid * nch, nch), :], didx)

        def descs(b, c):
            return (
                pltpu.make_async_copy(emb_hbm.at[sidx.at[c]], srows[b], sems[b][0]),
                pltpu.make_async_copy(emb_hbm.at[didx.at[c]], trows[b], sems[b][1]),
            )

        for b in range(NBUF):
            for d in descs(b, b):
                d.start()

        def compute(sr, tr, c):
            def group(g, _):
                e0 = g * 16
                for kk in range(16):
                    s = sr.at[e0 + kk]
                    t = tr.at[e0 + kk]
                    acc = (s[pl.ds(0, 16)] * t[pl.ds(0, 16)]
                           + s[pl.ds(16, 16)] * t[pl.ds(16, 16)]
                           + s[pl.ds(32, 16)] * t[pl.ds(32, 16)]
                           + s[pl.ds(48, 16)] * t[pl.ds(48, 16)])
                    part[kk, :] = acc
                tot = jnp.zeros((LANES,), jnp.float32)
                for j in range(16):
                    tot = tot + plsc.load_gather(
                        part, [lane, jnp.full((LANES,), j, jnp.int32)])
                outbuf.at[c][pl.ds(e0, 16)] = tot
                return 0

            lax.fori_loop(0, C // 16, group, 0)

        def macro(m, _):
            for b in range(NBUF):
                c = m * NBUF + b
                for d in descs(b, c):
                    d.wait()
                compute(srows[b], trows[b], c)

                @pl.when(c + NBUF < nch)
                def _():
                    for d in descs(b, c + NBUF):
                        d.start()

            return 0

        lax.fori_loop(0, nch // NBUF, macro, 0)
        pltpu.sync_copy(outbuf, out_hbm.at[pl.ds(wid * nch, nch), :])

    return k


def kernel(edge_index, emb1):
    n_nodes, dim = emb1.shape
    e = edge_index.shape[1]
    block = NW * C * NBUF
    e_pad = ((e + block - 1) // block) * block
    src = edge_index[0]
    dst = edge_index[1]
    if e_pad != e:
        pad = jnp.zeros((e_pad - e,), jnp.int32)
        src = jnp.concatenate([src, pad])
        dst = jnp.concatenate([dst, pad])
    nch = e_pad // (NW * C)
    src2d = src.reshape(NW * nch, C)
    dst2d = dst.reshape(NW * nch, C)
    out = _build_sc_kernel(e_pad, n_nodes, dim)(src2d, dst2d, emb1)
    return out.reshape(e_pad)[:e]


# speedup vs baseline: 1.7713x; 1.7713x over previous
import functools

import jax
import jax.numpy as jnp
from jax import lax
from jax.experimental import pallas as pl
from jax.experimental.pallas import tpu as pltpu
from jax.experimental.pallas import tpu_sc as plsc

NC = 2
NS = 16
NW = NC * NS
LANES = 16

C = 128
NBUF = 2


def _build_sc_kernel(e_pad: int, n_nodes: int, dim: int):
    assert dim == 64
    nch = e_pad // (NW * C)
    assert nch % NBUF == 0
    mesh = plsc.VectorSubcoreMesh(core_axis_name="c", subcore_axis_name="s")

    @functools.partial(
        pl.kernel,
        out_type=jax.ShapeDtypeStruct((NW * nch, C), jnp.float32),
        mesh=mesh,
        compiler_params=pltpu.CompilerParams(
            needs_layout_passes=False, use_tc_tiling_on_sc=False),
        scratch_types=[
            pltpu.VMEM((nch, C), jnp.int32),
            pltpu.VMEM((nch, C), jnp.int32),
        ] + [pltpu.VMEM((C, 64), jnp.bfloat16) for _ in range(2 * NBUF)]
          + [pltpu.VMEM((16, LANES), jnp.float32),
             pltpu.VMEM((nch, C), jnp.float32)]
          + [pltpu.SemaphoreType.DMA for _ in range(2 * NBUF)],
    )
    def k(src_hbm, dst_hbm, emb_hbm, out_hbm, sidx, didx, *rest):
        rowbufs = rest[:2 * NBUF]
        part, outbuf = rest[2 * NBUF], rest[2 * NBUF + 1]
        sembufs = rest[2 * NBUF + 2:]
        wid = lax.axis_index("s") * NC + lax.axis_index("c")
        lane = lax.iota(jnp.int32, LANES)
        srows = tuple(rowbufs[2 * b] for b in range(NBUF))
        trows = tuple(rowbufs[2 * b + 1] for b in range(NBUF))
        sems = tuple((sembufs[2 * b], sembufs[2 * b + 1]) for b in range(NBUF))

        pltpu.sync_copy(src_hbm.at[pl.ds(wid * nch, nch), :], sidx)
        pltpu.sync_copy(dst_hbm.at[pl.ds(wid * nch, nch), :], didx)

        def descs(b, c):
            return (
                pltpu.make_async_copy(emb_hbm.at[sidx.at[c]], srows[b], sems[b][0]),
                pltpu.make_async_copy(emb_hbm.at[didx.at[c]], trows[b], sems[b][1]),
            )

        for b in range(NBUF):
            for d in descs(b, b):
                d.start()

        def compute(sr, tr, c):
            def group(g, _):
                e0 = g * 16
                for kk in range(16):
                    s = sr.at[e0 + kk]
                    t = tr.at[e0 + kk]
                    sa, sb = plsc.unpack(
                        s[pl.ds(0, 32)], format=plsc.PackFormat.INTERLEAVED)
                    sc_, sd = plsc.unpack(
                        s[pl.ds(32, 32)], format=plsc.PackFormat.INTERLEAVED)
                    ta, tb = plsc.unpack(
                        t[pl.ds(0, 32)], format=plsc.PackFormat.INTERLEAVED)
                    tc_, td = plsc.unpack(
                        t[pl.ds(32, 32)], format=plsc.PackFormat.INTERLEAVED)
                    acc = sa * ta + sb * tb + sc_ * tc_ + sd * td
                    part[kk, :] = acc
                tot = jnp.zeros((LANES,), jnp.float32)
                for j in range(16):
                    tot = tot + plsc.load_gather(
                        part, [lane, jnp.full((LANES,), j, jnp.int32)])
                outbuf.at[c][pl.ds(e0, 16)] = tot
                return 0

            lax.fori_loop(0, C // 16, group, 0)

        def macro(m, _):
            for b in range(NBUF):
                c = m * NBUF + b
                for d in descs(b, c):
                    d.wait()
                compute(srows[b], trows[b], c)

                @pl.when(c + NBUF < nch)
                def _():
                    for d in descs(b, c + NBUF):
                        d.start()

            return 0

        lax.fori_loop(0, nch // NBUF, macro, 0)
        pltpu.sync_copy(outbuf, out_hbm.at[pl.ds(wid * nch, nch), :])

    return k


def kernel(edge_index, emb1):
    n_nodes, dim = emb1.shape
    e = edge_index.shape[1]
    block = NW * C * NBUF
    e_pad = ((e + block - 1) // block) * block
    src = edge_index[0]
    dst = edge_index[1]
    if e_pad != e:
        pad = jnp.zeros((e_pad - e,), jnp.int32)
        src = jnp.concatenate([src, pad])
        dst = jnp.concatenate([dst, pad])
    nch = e_pad // (NW * C)
    src2d = src.reshape(NW * nch, C)
    dst2d = dst.reshape(NW * nch, C)
    out = _build_sc_kernel(e_pad, n_nodes, dim)(
        src2d, dst2d, emb1.astype(jnp.bfloat16))
    return out.reshape(e_pad)[:e]


# speedup vs baseline: 1.7757x; 1.0025x over previous
import functools

import jax
import jax.numpy as jnp
from jax import lax
from jax.experimental import pallas as pl
from jax.experimental.pallas import tpu as pltpu
from jax.experimental.pallas import tpu_sc as plsc

NC = 2
NS = 16
NW = NC * NS
LANES = 16

C = 128
NBUF = 2


def _build_sc_kernel(e_pad: int, n_nodes: int, dim: int):
    assert dim == 64
    nch = e_pad // (NW * C)
    assert nch % NBUF == 0
    mesh = plsc.VectorSubcoreMesh(core_axis_name="c", subcore_axis_name="s")

    @functools.partial(
        pl.kernel,
        out_type=jax.ShapeDtypeStruct((NW * nch, C), jnp.float32),
        mesh=mesh,
        compiler_params=pltpu.CompilerParams(
            needs_layout_passes=False, use_tc_tiling_on_sc=False),
        scratch_types=[
            pltpu.VMEM((nch, C), jnp.int32),
            pltpu.VMEM((nch, C), jnp.int32),
        ] + [pltpu.VMEM((C, 64), jnp.bfloat16) for _ in range(2 * NBUF)]
          + [pltpu.VMEM((16, LANES), jnp.float32),
             pltpu.VMEM((nch, C), jnp.float32)]
          + [pltpu.SemaphoreType.DMA for _ in range(2 * NBUF)],
    )
    def k(src_hbm, dst_hbm, emb_hbm, out_hbm, sidx, didx, *rest):
        rowbufs = rest[:2 * NBUF]
        part, outbuf = rest[2 * NBUF], rest[2 * NBUF + 1]
        sembufs = rest[2 * NBUF + 2:]
        wid = lax.axis_index("s") * NC + lax.axis_index("c")
        lane = lax.iota(jnp.int32, LANES)
        srows = tuple(rowbufs[2 * b] for b in range(NBUF))
        trows = tuple(rowbufs[2 * b + 1] for b in range(NBUF))
        sems = tuple((sembufs[2 * b], sembufs[2 * b + 1]) for b in range(NBUF))

        pltpu.sync_copy(src_hbm.at[pl.ds(wid * nch, nch), :], sidx)
        pltpu.sync_copy(dst_hbm.at[pl.ds(wid * nch, nch), :], didx)

        def descs(b, c):
            return (
                pltpu.make_async_copy(emb_hbm.at[sidx.at[c]], srows[b], sems[b][0]),
                pltpu.make_async_copy(emb_hbm.at[didx.at[c]], trows[b], sems[b][1]),
            )

        for b in range(NBUF):
            for d in descs(b, b):
                d.start()

        def compute(sr, tr, c):
            def group(g, _):
                e0 = g * 16
                for kk in range(16):
                    s = sr.at[e0 + kk]
                    t = tr.at[e0 + kk]
                    acc32 = (s[pl.ds(0, 32)] * t[pl.ds(0, 32)]
                             + s[pl.ds(32, 32)] * t[pl.ds(32, 32)])
                    hi, lo = plsc.unpack(
                        acc32, format=plsc.PackFormat.INTERLEAVED)
                    part[kk, :] = hi + lo
                tot = jnp.zeros((LANES,), jnp.float32)
                for j in range(16):
                    tot = tot + plsc.load_gather(
                        part, [lane, jnp.full((LANES,), j, jnp.int32)])
                outbuf.at[c][pl.ds(e0, 16)] = tot
                return 0

            lax.fori_loop(0, C // 16, group, 0)

        def macro(m, _):
            for b in range(NBUF):
                c = m * NBUF + b
                for d in descs(b, c):
                    d.wait()
                compute(srows[b], trows[b], c)

                @pl.when(c + NBUF < nch)
                def _():
                    for d in descs(b, c + NBUF):
                        d.start()

            return 0

        lax.fori_loop(0, nch // NBUF, macro, 0)
        pltpu.sync_copy(outbuf, out_hbm.at[pl.ds(wid * nch, nch), :])

    return k


def kernel(edge_index, emb1):
    n_nodes, dim = emb1.shape
    e = edge_index.shape[1]
    block = NW * C * NBUF
    e_pad = ((e + block - 1) // block) * block
    src = edge_index[0]
    dst = edge_index[1]
    if e_pad != e:
        pad = jnp.zeros((e_pad - e,), jnp.int32)
        src = jnp.concatenate([src, pad])
        dst = jnp.concatenate([dst, pad])
    nch = e_pad // (NW * C)
    src2d = src.reshape(NW * nch, C)
    dst2d = dst.reshape(NW * nch, C)
    out = _build_sc_kernel(e_pad, n_nodes, dim)(
        src2d, dst2d, emb1.astype(jnp.bfloat16))
    return out.reshape(e_pad)[:e]


# speedup vs baseline: 1.8655x; 1.0506x over previous
import functools

import jax
import jax.numpy as jnp
from jax import lax
from jax.experimental import pallas as pl
from jax.experimental.pallas import tpu as pltpu
from jax.experimental.pallas import tpu_sc as plsc

NC = 2
NS = 16
NW = NC * NS
LANES = 16

C = 128
RBUF = 2
IBUF = 4
OBUF = 2
UNROLL = 4


def _build_sc_kernel(e_pad: int, n_nodes: int, dim: int):
    assert dim == 64
    nch = e_pad // (NW * C)
    assert nch % UNROLL == 0 and nch >= 2 * UNROLL
    mesh = plsc.VectorSubcoreMesh(core_axis_name="c", subcore_axis_name="s")

    @functools.partial(
        pl.kernel,
        out_type=jax.ShapeDtypeStruct((NW * nch, C), jnp.float32),
        mesh=mesh,
        compiler_params=pltpu.CompilerParams(
            needs_layout_passes=False, use_tc_tiling_on_sc=False),
        scratch_types=(
            [pltpu.VMEM((2, C), jnp.int32) for _ in range(IBUF)]
            + [pltpu.VMEM((C, 64), jnp.bfloat16) for _ in range(2 * RBUF)]
            + [pltpu.VMEM((C,), jnp.float32) for _ in range(OBUF)]
            + [pltpu.VMEM((16, LANES), jnp.float32),
               pltpu.VMEM_SHARED((n_nodes, 64), jnp.bfloat16)]
            + [pltpu.SemaphoreType.DMA for _ in range(IBUF + 2 * RBUF + OBUF)]
        ),
    )
    def k(cidx_hbm, emb_hbm, out_hbm, *rest):
        idxs = rest[:IBUF]
        rowbufs = rest[IBUF:IBUF + 2 * RBUF]
        outvs = rest[IBUF + 2 * RBUF:IBUF + 2 * RBUF + OBUF]
        part = rest[IBUF + 2 * RBUF + OBUF]
        table_sh = rest[IBUF + 2 * RBUF + OBUF + 1]
        sems = rest[IBUF + 2 * RBUF + OBUF + 2:]
        sem_i = sems[:IBUF]
        sem_r = sems[IBUF:IBUF + 2 * RBUF]
        sem_o = sems[IBUF + 2 * RBUF:]
        srows = tuple(rowbufs[2 * b] for b in range(RBUF))
        trows = tuple(rowbufs[2 * b + 1] for b in range(RBUF))

        wid = lax.axis_index("s") * NC + lax.axis_index("c")
        lane = lax.iota(jnp.int32, LANES)

        sub = lax.axis_index("s")
        rpt = n_nodes // NS
        rem = n_nodes - rpt * NS
        pltpu.sync_copy(emb_hbm.at[pl.ds(sub * rpt, rpt), :],
                        table_sh.at[pl.ds(sub * rpt, rpt), :])

        @pl.when(sub == 0)
        def _():
            if rem:
                pltpu.sync_copy(emb_hbm.at[pl.ds(NS * rpt, rem), :],
                                table_sh.at[pl.ds(NS * rpt, rem), :])

        def idx_desc(c, j):
            return pltpu.make_async_copy(
                cidx_hbm.at[wid * nch + c], idxs[j], sem_i[j])

        def gather_descs(c, b, j):
            return (
                pltpu.make_async_copy(
                    table_sh.at[idxs[j].at[0]], srows[b], sem_r[2 * b]),
                pltpu.make_async_copy(
                    table_sh.at[idxs[j].at[1]], trows[b], sem_r[2 * b + 1]),
            )

        def out_desc(c, b):
            return pltpu.make_async_copy(
                outvs[b], out_hbm.at[wid * nch + c], sem_o[b])

        for j in range(IBUF):
            idx_desc(j, j).start()

        plsc.subcore_barrier()

        for c in range(RBUF):
            idx_desc(c, c).wait()
            for d in gather_descs(c, c, c):
                d.start()

        def compute(sr, tr, outv):
            def group(g, _):
                e0 = g * 16
                for kk in range(16):
                    s = sr.at[e0 + kk]
                    t = tr.at[e0 + kk]
                    acc32 = (s[pl.ds(0, 32)] * t[pl.ds(0, 32)]
                             + s[pl.ds(32, 32)] * t[pl.ds(32, 32)])
                    hi, lo = plsc.unpack(
                        acc32, format=plsc.PackFormat.INTERLEAVED)
                    part[kk, :] = hi + lo
                tot = jnp.zeros((LANES,), jnp.float32)
                for j in range(16):
                    tot = tot + plsc.load_gather(
                        part, [lane, jnp.full((LANES,), j, jnp.int32)])
                outv[pl.ds(e0, 16)] = tot
                return 0

            lax.fori_loop(0, C // 16, group, 0)

        def macro(m, _):
            for u in range(UNROLL):
                c = m * UNROLL + u
                b = u % RBUF
                jo = u % OBUF
                for d in gather_descs(c, b, u):
                    d.wait()

                @pl.when(c >= OBUF)
                def _():
                    out_desc(c - OBUF, jo).wait()

                compute(srows[b], trows[b], outvs[jo])
                out_desc(c, jo).start()

                @pl.when(c + IBUF < nch)
                def _():
                    idx_desc(c + IBUF, u % IBUF).start()

                @pl.when(c + RBUF < nch)
                def _():
                    idx_desc(c + RBUF, (u + RBUF) % IBUF).wait()
                    for d in gather_descs(c + RBUF, b, (u + RBUF) % IBUF):
                        d.start()

            return 0

        lax.fori_loop(0, nch // UNROLL, macro, 0)
        for c in range(nch - OBUF, nch):
            out_desc(c, c % OBUF).wait()

    return k


def kernel(edge_index, emb1):
    n_nodes, dim = emb1.shape
    e = edge_index.shape[1]
    block = NW * C * UNROLL
    e_pad = ((e + block - 1) // block) * block
    src = edge_index[0]
    dst = edge_index[1]
    if e_pad != e:
        pad = jnp.zeros((e_pad - e,), jnp.int32)
        src = jnp.concatenate([src, pad])
        dst = jnp.concatenate([dst, pad])
    nch = e_pad // (NW * C)
    cidx = jnp.stack([src.reshape(NW * nch, C), dst.reshape(NW * nch, C)],
                     axis=1)
    out = _build_sc_kernel(e_pad, n_nodes, dim)(
        cidx, emb1.astype(jnp.bfloat16))
    return out.reshape(e_pad)[:e]


# speedup vs baseline: 2.6491x; 1.4200x over previous
import functools

import jax
import jax.numpy as jnp
from jax import lax
from jax.experimental import pallas as pl
from jax.experimental.pallas import tpu as pltpu
from jax.experimental.pallas import tpu_sc as plsc

NC = 2
NS = 16
NW = NC * NS
LANES = 16

C = 128
RBUF = 2
IBUF = 4
OBUF = 2
UNROLL = 4


def _build_sc_kernel(e_pad: int, n_nodes: int, dim: int):
    assert dim == 64
    nch = e_pad // (NW * C)
    assert nch % UNROLL == 0 and nch >= 2 * UNROLL
    mesh = plsc.VectorSubcoreMesh(core_axis_name="c", subcore_axis_name="s")

    @functools.partial(
        pl.kernel,
        out_type=jax.ShapeDtypeStruct((NW * nch, C), jnp.float32),
        mesh=mesh,
        compiler_params=pltpu.CompilerParams(
            needs_layout_passes=False, use_tc_tiling_on_sc=False),
        scratch_types=(
            [pltpu.VMEM((2, C), jnp.int32) for _ in range(IBUF)]
            + [pltpu.VMEM((C, 64), jnp.bfloat16) for _ in range(2 * RBUF)]
            + [pltpu.VMEM((C,), jnp.float32) for _ in range(OBUF)]
            + [pltpu.VMEM((16, LANES), jnp.float32),
               pltpu.VMEM_SHARED((n_nodes, 64), jnp.bfloat16)]
            + [pltpu.SemaphoreType.DMA for _ in range(IBUF + 2 * RBUF + OBUF)]
        ),
    )
    def k(cidx_hbm, emb_hbm, out_hbm, *rest):
        idxs = rest[:IBUF]
        rowbufs = rest[IBUF:IBUF + 2 * RBUF]
        outvs = rest[IBUF + 2 * RBUF:IBUF + 2 * RBUF + OBUF]
        part = rest[IBUF + 2 * RBUF + OBUF]
        table_sh = rest[IBUF + 2 * RBUF + OBUF + 1]
        sems = rest[IBUF + 2 * RBUF + OBUF + 2:]
        sem_i = sems[:IBUF]
        sem_r = sems[IBUF:IBUF + 2 * RBUF]
        sem_o = sems[IBUF + 2 * RBUF:]
        srows = tuple(rowbufs[2 * b] for b in range(RBUF))
        trows = tuple(rowbufs[2 * b + 1] for b in range(RBUF))

        wid = lax.axis_index("s") * NC + lax.axis_index("c")
        lane = lax.iota(jnp.int32, LANES)

        sub = lax.axis_index("s")
        rpt = n_nodes // NS
        rem = n_nodes - rpt * NS
        pltpu.sync_copy(emb_hbm.at[pl.ds(sub * rpt, rpt), :],
                        table_sh.at[pl.ds(sub * rpt, rpt), :])

        @pl.when(sub == 0)
        def _():
            if rem:
                pltpu.sync_copy(emb_hbm.at[pl.ds(NS * rpt, rem), :],
                                table_sh.at[pl.ds(NS * rpt, rem), :])

        def idx_desc(c, j):
            return pltpu.make_async_copy(
                cidx_hbm.at[wid * nch + c], idxs[j], sem_i[j])

        def gather_descs(c, b, j):
            return (
                pltpu.make_async_copy(
                    table_sh.at[idxs[j].at[0]], srows[b], sem_r[2 * b]),
                pltpu.make_async_copy(
                    table_sh.at[idxs[j].at[1]], trows[b], sem_r[2 * b + 1]),
            )

        def out_desc(c, b):
            return pltpu.make_async_copy(
                outvs[b], out_hbm.at[wid * nch + c], sem_o[b])

        for j in range(IBUF):
            idx_desc(j, j).start()

        plsc.subcore_barrier()

        for c in range(RBUF):
            idx_desc(c, c).wait()
            for d in gather_descs(c, c, c):
                d.start()

        def compute(sr, tr, outv):
            def group(g, _):
                e0 = g * 16
                for kk in range(0, 16, 4):
                    sl, sh, tl, th = [], [], [], []
                    for i in range(4):
                        s = sr.at[e0 + kk + i]
                        t = tr.at[e0 + kk + i]
                        sl.append(s[pl.ds(0, 32)])
                        sh.append(s[pl.ds(32, 32)])
                        tl.append(t[pl.ds(0, 32)])
                        th.append(t[pl.ds(32, 32)])
                    acc32 = [sl[i] * tl[i] + sh[i] * th[i] for i in range(4)]
                    for i in range(4):
                        hi, lo = plsc.unpack(
                            acc32[i], format=plsc.PackFormat.INTERLEAVED)
                        part[kk + i, :] = hi + lo
                tot = jnp.zeros((LANES,), jnp.float32)
                for j in range(16):
                    tot = tot + plsc.load_gather(
                        part, [lane, jnp.full((LANES,), j, jnp.int32)])
                outv[pl.ds(e0, 16)] = tot
                return 0

            lax.fori_loop(0, C // 16, group, 0)

        def macro(m, _):
            for u in range(UNROLL):
                c = m * UNROLL + u
                b = u % RBUF
                jo = u % OBUF
                for d in gather_descs(c, b, u):
                    d.wait()

                @pl.when(c >= OBUF)
                def _():
                    out_desc(c - OBUF, jo).wait()

                compute(srows[b], trows[b], outvs[jo])
                out_desc(c, jo).start()

                @pl.when(c + IBUF < nch)
                def _():
                    idx_desc(c + IBUF, u % IBUF).start()

                @pl.when(c + RBUF < nch)
                def _():
                    idx_desc(c + RBUF, (u + RBUF) % IBUF).wait()
                    for d in gather_descs(c + RBUF, b, (u + RBUF) % IBUF):
                        d.start()

            return 0

        lax.fori_loop(0, nch // UNROLL, macro, 0)
        for c in range(nch - OBUF, nch):
            out_desc(c, c % OBUF).wait()

    return k


def kernel(edge_index, emb1):
    n_nodes, dim = emb1.shape
    e = edge_index.shape[1]
    block = NW * C * UNROLL
    e_pad = ((e + block - 1) // block) * block
    src = edge_index[0]
    dst = edge_index[1]
    if e_pad != e:
        pad = jnp.zeros((e_pad - e,), jnp.int32)
        src = jnp.concatenate([src, pad])
        dst = jnp.concatenate([dst, pad])
    nch = e_pad // (NW * C)
    cidx = jnp.stack([src.reshape(NW * nch, C), dst.reshape(NW * nch, C)],
                     axis=1)
    out = _build_sc_kernel(e_pad, n_nodes, dim)(
        cidx, emb1.astype(jnp.bfloat16))
    return out.reshape(e_pad)[:e]


# speedup vs baseline: 2.8267x; 1.0670x over previous
import functools

import jax
import jax.numpy as jnp
from jax import lax
from jax.experimental import pallas as pl
from jax.experimental.pallas import tpu as pltpu
from jax.experimental.pallas import tpu_sc as plsc

NC = 2
NS = 16
NW = NC * NS
LANES = 16

C = 128
RBUF = 2
IBUF = 4
OBUF = 2
UNROLL = 4


def _build_sc_kernel(e_pad: int, n_nodes: int, dim: int):
    assert dim == 64
    nch = e_pad // (NW * C)
    assert nch % UNROLL == 0 and nch >= 2 * UNROLL
    mesh = plsc.VectorSubcoreMesh(core_axis_name="c", subcore_axis_name="s")

    @functools.partial(
        pl.kernel,
        out_type=jax.ShapeDtypeStruct((NW * nch, C), jnp.float32),
        mesh=mesh,
        compiler_params=pltpu.CompilerParams(
            needs_layout_passes=False, use_tc_tiling_on_sc=False),
        scratch_types=(
            [pltpu.VMEM((2, C), jnp.int32) for _ in range(IBUF)]
            + [pltpu.VMEM((C, 64), jnp.bfloat16) for _ in range(2 * RBUF)]
            + [pltpu.VMEM((C,), jnp.float32) for _ in range(OBUF)]
            + [pltpu.VMEM((16, LANES), jnp.float32),
               pltpu.VMEM_SHARED((n_nodes, 64), jnp.bfloat16)]
            + [pltpu.SemaphoreType.DMA for _ in range(IBUF + 2 * RBUF + OBUF)]
        ),
    )
    def k(cidx_hbm, emb_hbm, out_hbm, *rest):
        idxs = rest[:IBUF]
        rowbufs = rest[IBUF:IBUF + 2 * RBUF]
        outvs = rest[IBUF + 2 * RBUF:IBUF + 2 * RBUF + OBUF]
        part = rest[IBUF + 2 * RBUF + OBUF]
        table_sh = rest[IBUF + 2 * RBUF + OBUF + 1]
        sems = rest[IBUF + 2 * RBUF + OBUF + 2:]
        sem_i = sems[:IBUF]
        sem_r = sems[IBUF:IBUF + 2 * RBUF]
        sem_o = sems[IBUF + 2 * RBUF:]
        srows = tuple(rowbufs[2 * b] for b in range(RBUF))
        trows = tuple(rowbufs[2 * b + 1] for b in range(RBUF))

        wid = lax.axis_index("s") * NC + lax.axis_index("c")
        lane = lax.iota(jnp.int32, LANES)

        sub = lax.axis_index("s")
        rpt = n_nodes // NS
        rem = n_nodes - rpt * NS
        pltpu.sync_copy(emb_hbm.at[pl.ds(sub * rpt, rpt), :],
                        table_sh.at[pl.ds(sub * rpt, rpt), :])

        @pl.when(sub == 0)
        def _():
            if rem:
                pltpu.sync_copy(emb_hbm.at[pl.ds(NS * rpt, rem), :],
                                table_sh.at[pl.ds(NS * rpt, rem), :])

        def idx_desc(c, j):
            return pltpu.make_async_copy(
                cidx_hbm.at[wid * nch + c], idxs[j], sem_i[j])

        def gather_descs(c, b, j):
            return (
                pltpu.make_async_copy(
                    table_sh.at[idxs[j].at[0]], srows[b], sem_r[2 * b]),
                pltpu.make_async_copy(
                    table_sh.at[idxs[j].at[1]], trows[b], sem_r[2 * b + 1]),
            )

        def out_desc(c, b):
            return pltpu.make_async_copy(
                outvs[b], out_hbm.at[wid * nch + c], sem_o[b])

        for j in range(IBUF):
            idx_desc(j, j).start()

        plsc.subcore_barrier()

        for c in range(RBUF):
            idx_desc(c, c).wait()
            for d in gather_descs(c, c, c):
                d.start()

        def compute(sr, tr, outv):
            def group(g, _):
                e0 = g * 16
                for kk in range(0, 16, 8):
                    sl, sh, tl, th = [], [], [], []
                    for i in range(8):
                        s = sr.at[e0 + kk + i]
                        t = tr.at[e0 + kk + i]
                        sl.append(s[pl.ds(0, 32)])
                        sh.append(s[pl.ds(32, 32)])
                        tl.append(t[pl.ds(0, 32)])
                        th.append(t[pl.ds(32, 32)])
                    acc32 = [sl[i] * tl[i] + sh[i] * th[i] for i in range(8)]
                    for i in range(8):
                        hi, lo = plsc.unpack(
                            acc32[i], format=plsc.PackFormat.INTERLEAVED)
                        part[kk + i, :] = hi + lo
                tot = jnp.zeros((LANES,), jnp.float32)
                for j in range(16):
                    tot = tot + plsc.load_gather(
                        part, [lane, jnp.full((LANES,), j, jnp.int32)])
                outv[pl.ds(e0, 16)] = tot
                return 0

            lax.fori_loop(0, C // 16, group, 0)

        def macro(m, _):
            for u in range(UNROLL):
                c = m * UNROLL + u
                b = u % RBUF
                jo = u % OBUF
                for d in gather_descs(c, b, u):
                    d.wait()

                @pl.when(c >= OBUF)
                def _():
                    out_desc(c - OBUF, jo).wait()

                compute(srows[b], trows[b], outvs[jo])
                out_desc(c, jo).start()

                @pl.when(c + IBUF < nch)
                def _():
                    idx_desc(c + IBUF, u % IBUF).start()

                @pl.when(c + RBUF < nch)
                def _():
                    idx_desc(c + RBUF, (u + RBUF) % IBUF).wait()
                    for d in gather_descs(c + RBUF, b, (u + RBUF) % IBUF):
                        d.start()

            return 0

        lax.fori_loop(0, nch // UNROLL, macro, 0)
        for c in range(nch - OBUF, nch):
            out_desc(c, c % OBUF).wait()

    return k


def kernel(edge_index, emb1):
    n_nodes, dim = emb1.shape
    e = edge_index.shape[1]
    block = NW * C * UNROLL
    e_pad = ((e + block - 1) // block) * block
    src = edge_index[0]
    dst = edge_index[1]
    if e_pad != e:
        pad = jnp.zeros((e_pad - e,), jnp.int32)
        src = jnp.concatenate([src, pad])
        dst = jnp.concatenate([dst, pad])
    nch = e_pad // (NW * C)
    cidx = jnp.stack([src.reshape(NW * nch, C), dst.reshape(NW * nch, C)],
                     axis=1)
    out = _build_sc_kernel(e_pad, n_nodes, dim)(
        cidx, emb1.astype(jnp.bfloat16))
    return out.reshape(e_pad)[:e]


# speedup vs baseline: 3.8433x; 1.3597x over previous
import functools

import jax
import jax.numpy as jnp
from jax import lax
from jax.experimental import pallas as pl
from jax.experimental.pallas import tpu as pltpu
from jax.experimental.pallas import tpu_sc as plsc

NC = 2
NS = 16
NW = NC * NS
LANES = 16

C = 128
RBUF = 2
IBUF = 4
OBUF = 2
UNROLL = 4


def _build_sc_kernel(e_pad: int, n_nodes: int, dim: int):
    assert dim == 64
    nch = e_pad // (NW * C)
    assert nch % UNROLL == 0 and nch >= 2 * UNROLL
    mesh = plsc.VectorSubcoreMesh(core_axis_name="c", subcore_axis_name="s")

    @functools.partial(
        pl.kernel,
        out_type=jax.ShapeDtypeStruct((NW * nch, C), jnp.float32),
        mesh=mesh,
        compiler_params=pltpu.CompilerParams(
            needs_layout_passes=False, use_tc_tiling_on_sc=False),
        scratch_types=(
            [pltpu.VMEM((2, C), jnp.int32) for _ in range(IBUF)]
            + [pltpu.VMEM((C, 64), jnp.bfloat16) for _ in range(2 * RBUF)]
            + [pltpu.VMEM((C,), jnp.float32) for _ in range(OBUF)]
            + [pltpu.VMEM((16, LANES), jnp.float32),
               pltpu.VMEM_SHARED((n_nodes, 64), jnp.bfloat16)]
            + [pltpu.SemaphoreType.DMA for _ in range(IBUF + 2 * RBUF + OBUF)]
        ),
    )
    def k(cidx_hbm, emb_hbm, out_hbm, *rest):
        idxs = rest[:IBUF]
        rowbufs = rest[IBUF:IBUF + 2 * RBUF]
        outvs = rest[IBUF + 2 * RBUF:IBUF + 2 * RBUF + OBUF]
        part = rest[IBUF + 2 * RBUF + OBUF]
        table_sh = rest[IBUF + 2 * RBUF + OBUF + 1]
        sems = rest[IBUF + 2 * RBUF + OBUF + 2:]
        sem_i = sems[:IBUF]
        sem_r = sems[IBUF:IBUF + 2 * RBUF]
        sem_o = sems[IBUF + 2 * RBUF:]
        srows = tuple(rowbufs[2 * b] for b in range(RBUF))
        trows = tuple(rowbufs[2 * b + 1] for b in range(RBUF))

        wid = lax.axis_index("s") * NC + lax.axis_index("c")
        lane = lax.iota(jnp.int32, LANES)

        sub = lax.axis_index("s")
        rpt = n_nodes // NS
        rem = n_nodes - rpt * NS
        pltpu.sync_copy(emb_hbm.at[pl.ds(sub * rpt, rpt), :],
                        table_sh.at[pl.ds(sub * rpt, rpt), :])

        @pl.when(sub == 0)
        def _():
            if rem:
                pltpu.sync_copy(emb_hbm.at[pl.ds(NS * rpt, rem), :],
                                table_sh.at[pl.ds(NS * rpt, rem), :])

        def idx_desc(c, j):
            return pltpu.make_async_copy(
                cidx_hbm.at[wid * nch + c], idxs[j], sem_i[j])

        def gather_descs(c, b, j):
            return (
                pltpu.make_async_copy(
                    table_sh.at[idxs[j].at[0]], srows[b], sem_r[2 * b]),
                pltpu.make_async_copy(
                    table_sh.at[idxs[j].at[1]], trows[b], sem_r[2 * b + 1]),
            )

        def out_desc(c, b):
            return pltpu.make_async_copy(
                outvs[b], out_hbm.at[wid * nch + c], sem_o[b])

        for j in range(IBUF):
            idx_desc(j, j).start()

        plsc.subcore_barrier()

        for c in range(RBUF):
            idx_desc(c, c).wait()
            for d in gather_descs(c, c, c):
                d.start()

        def compute(sr, tr, outv):
            def group(g, _):
                e0 = g * 16
                tot = jnp.zeros((LANES,), jnp.float32)
                for kk in range(0, 16, 8):
                    sl, sh, tl, th = [], [], [], []
                    for i in range(8):
                        s = sr.at[e0 + kk + i]
                        t = tr.at[e0 + kk + i]
                        sl.append(s[pl.ds(0, 32)])
                        sh.append(s[pl.ds(32, 32)])
                        tl.append(t[pl.ds(0, 32)])
                        th.append(t[pl.ds(32, 32)])
                    acc32 = [sl[i] * tl[i] + sh[i] * th[i] for i in range(8)]
                    for i in range(8):
                        hi, lo = plsc.unpack(
                            acc32[i], format=plsc.PackFormat.INTERLEAVED)
                        tot = jnp.where(lane == kk + i,
                                        jnp.sum(hi + lo, axis=0), tot)
                outv[pl.ds(e0, 16)] = tot
                return 0

            lax.fori_loop(0, C // 16, group, 0)

        def macro(m, _):
            for u in range(UNROLL):
                c = m * UNROLL + u
                b = u % RBUF
                jo = u % OBUF
                for d in gather_descs(c, b, u):
                    d.wait()

                @pl.when(c >= OBUF)
                def _():
                    out_desc(c - OBUF, jo).wait()

                compute(srows[b], trows[b], outvs[jo])
                out_desc(c, jo).start()

                @pl.when(c + IBUF < nch)
                def _():
                    idx_desc(c + IBUF, u % IBUF).start()

                @pl.when(c + RBUF < nch)
                def _():
                    idx_desc(c + RBUF, (u + RBUF) % IBUF).wait()
                    for d in gather_descs(c + RBUF, b, (u + RBUF) % IBUF):
                        d.start()

            return 0

        lax.fori_loop(0, nch // UNROLL, macro, 0)
        for c in range(nch - OBUF, nch):
            out_desc(c, c % OBUF).wait()

    return k


def kernel(edge_index, emb1):
    n_nodes, dim = emb1.shape
    e = edge_index.shape[1]
    block = NW * C * UNROLL
    e_pad = ((e + block - 1) // block) * block
    src = edge_index[0]
    dst = edge_index[1]
    if e_pad != e:
        pad = jnp.zeros((e_pad - e,), jnp.int32)
        src = jnp.concatenate([src, pad])
        dst = jnp.concatenate([dst, pad])
    nch = e_pad // (NW * C)
    cidx = jnp.stack([src.reshape(NW * nch, C), dst.reshape(NW * nch, C)],
                     axis=1)
    out = _build_sc_kernel(e_pad, n_nodes, dim)(
        cidx, emb1.astype(jnp.bfloat16))
    return out.reshape(e_pad)[:e]
